# Initial kernel scaffold; baseline (speedup 1.0000x reference)
import jax
import jax.numpy as jnp
from jax.experimental import pallas as pl


def kernel(u, i, edges, entitys_w, users_w, W1, b1, W2, b2):
    raise NotImplementedError("write your pallas kernel here")


# speedup vs baseline: 12.7498x; 12.7498x over previous
import functools

import jax
import jax.numpy as jnp
from jax import lax
from jax.experimental import pallas as pl
from jax.experimental.pallas import tpu as pltpu
from jax.experimental.pallas import tpu_sc as plsc

N = 10000
E = 320000
D = 128
B = 4096
NC = 2
NS = 16
NW = NC * NS
EPT = E // NW
K = 80
CH = EPT // K
BPT = B // NW

_mesh = plsc.VectorSubcoreMesh(core_axis_name="c", subcore_axis_name="s")


def _per_tile_rows(s, body):
    @pl.when(s < NS - 1)
    def _():
        body(s * 624, 624)

    @pl.when(s == NS - 1)
    def _():
        body(624 * (NS - 1), 640)


_EB = 4000


def _tc_deg_body(dst_ref, hist_ref):
    step = pl.program_id(0)
    d = dst_ref[...]
    r = d >> 7
    c = d & 127
    ohr = (r == lax.broadcasted_iota(jnp.int32, (1, 80), 1)).astype(jnp.bfloat16)
    ohc = (c == lax.broadcasted_iota(jnp.int32, (1, D), 1)).astype(jnp.bfloat16)
    h = lax.dot_general(ohr, ohc, (((0,), (0,)), ((), ())),
                        preferred_element_type=jnp.float32)

    @pl.when(step == 0)
    def _():
        hist_ref[...] = h

    @pl.when(step > 0)
    def _():
        hist_ref[...] += h


_tc_deg = pl.pallas_call(
    _tc_deg_body,
    grid=(E // _EB,),
    in_specs=[pl.BlockSpec((_EB, 1), lambda k: (k, 0))],
    out_specs=pl.BlockSpec((80, D), lambda k: (0, 0)),
    out_shape=jax.ShapeDtypeStruct((80, D), jnp.float32),
)


@functools.partial(
    pl.kernel,
    out_type=jax.ShapeDtypeStruct((NC, N, D), jnp.float32),
    mesh=_mesh,
    scratch_types=[
        pltpu.VMEM((CH, K), jnp.int32),
        pltpu.VMEM((CH, K), jnp.int32),
        pltpu.VMEM((K, D), jnp.float32),
        pltpu.VMEM_SHARED((N, D), jnp.float32),
        pltpu.SemaphoreType.DMA,
    ],
)
def _sc_scatter_full(src2d, dst2d, hp, z128,
                     accp_out,
                     srci_v, dsti_v, rows_v, acc_sh, sem):
    c = lax.axis_index("c")
    s = lax.axis_index("s")
    wid = c * NS + s

    @pl.when(c == 0)
    def _():
        _per_tile_rows(s, lambda o, n: pltpu.sync_copy(
            hp.at[pl.ds(o, n)], acc_sh.at[pl.ds(o, n)]))

    @pl.when(c == 1)
    def _():
        _per_tile_rows(s, lambda o, n: pltpu.sync_copy(
            z128.at[pl.ds(o, n)], acc_sh.at[pl.ds(o, n)]))

    pltpu.sync_copy(src2d.at[wid], srci_v)
    pltpu.sync_copy(dst2d.at[wid], dsti_v)
    plsc.subcore_barrier()

    @pl.loop(0, CH)
    def _(j):
        pltpu.async_copy(hp.at[srci_v.at[j]], rows_v, sem).wait()
        pltpu.sync_copy(rows_v, acc_sh.at[dsti_v.at[j]], add=True)

    plsc.subcore_barrier()
    _per_tile_rows(s, lambda o, n: pltpu.sync_copy(
        acc_sh.at[pl.ds(o, n)], accp_out.at[c, pl.ds(o, n)]))


@functools.partial(
    pl.kernel,
    out_type=(
        jax.ShapeDtypeStruct((NC, B, D), jnp.float32),
        jax.ShapeDtypeStruct((B, D), jnp.float32),
        jax.ShapeDtypeStruct((B, D), jnp.float32),
    ),
    mesh=_mesh,
    scratch_types=[
        pltpu.VMEM((BPT,), jnp.int32),
        pltpu.VMEM((BPT, D), jnp.float32),
        pltpu.SemaphoreType.DMA,
    ],
)
def _sc_gather_items(a0, a1, dis, users_w, i2d, u2d,
                     g_out, disi_out, urows_out,
                     iidx_v, irow_v, sem):
    c = lax.axis_index("c")
    s = lax.axis_index("s")
    wid = c * NS + s
    pltpu.sync_copy(i2d.at[wid], iidx_v)
    pltpu.async_copy(a0.at[iidx_v], irow_v, sem).wait()
    pltpu.sync_copy(irow_v, g_out.at[0, pl.ds(wid * BPT, BPT)])
    pltpu.async_copy(a1.at[iidx_v], irow_v, sem).wait()
    pltpu.sync_copy(irow_v, g_out.at[1, pl.ds(wid * BPT, BPT)])
    pltpu.async_copy(dis.at[iidx_v], irow_v, sem).wait()
    pltpu.sync_copy(irow_v, disi_out.at[pl.ds(wid * BPT, BPT)])
    pltpu.sync_copy(u2d.at[wid], iidx_v)
    pltpu.async_copy(users_w.at[iidx_v], irow_v, sem).wait()
    pltpu.sync_copy(irow_v, urows_out.at[pl.ds(wid * BPT, BPT)])


_ROWS_B = 1000


def _tc_b_body(ent, w1, d0, h1p_out, dis_out):
    x = ent[...]
    n = jnp.sqrt(jnp.sum(x * x, axis=1, keepdims=True))
    x0 = x * jnp.minimum(1.0, 1.0 / (n + 1e-7))
    deg = d0[...] + 1.0
    dis = lax.rsqrt(deg)
    h1 = jnp.dot(x0, w1[...], preferred_element_type=jnp.float32)
    h1p_out[...] = h1 * dis
    dis_out[...] = jnp.broadcast_to(dis, (_ROWS_B, D))


_tc_b = pl.pallas_call(
    _tc_b_body,
    grid=(N // _ROWS_B,),
    in_specs=[
        pl.BlockSpec((_ROWS_B, D), lambda i: (i, 0)),
        pl.BlockSpec((D, D), lambda i: (0, 0)),
        pl.BlockSpec((_ROWS_B, 1), lambda i: (i, 0)),
    ],
    out_specs=(
        pl.BlockSpec((_ROWS_B, D), lambda i: (i, 0)),
        pl.BlockSpec((_ROWS_B, D), lambda i: (i, 0)),
    ),
    out_shape=(
        jax.ShapeDtypeStruct((N, D), jnp.float32),
        jax.ShapeDtypeStruct((N, D), jnp.float32),
    ),
)


def _tc_d_body(a0, a1, dis, b1, w2, h2p_out):
    x1 = jnp.maximum(dis[...] * (a0[...] + a1[...]) + b1[...], 0.0)
    h2 = jnp.dot(x1, w2[...], preferred_element_type=jnp.float32)
    h2p_out[...] = h2 * dis[...]


_tc_d = pl.pallas_call(
    _tc_d_body,
    grid=(N // _ROWS_B,),
    in_specs=[
        pl.BlockSpec((_ROWS_B, D), lambda i: (i, 0)),
        pl.BlockSpec((_ROWS_B, D), lambda i: (i, 0)),
        pl.BlockSpec((_ROWS_B, D), lambda i: (i, 0)),
        pl.BlockSpec((1, D), lambda i: (0, 0)),
        pl.BlockSpec((D, D), lambda i: (0, 0)),
    ],
    out_specs=pl.BlockSpec((_ROWS_B, D), lambda i: (i, 0)),
    out_shape=jax.ShapeDtypeStruct((N, D), jnp.float32),
)


_ROWS_F = 512


def _tc_f_body(ur, g0, g1, disi, b2, out):
    x = ur[...]
    n = jnp.sqrt(jnp.sum(x * x, axis=1, keepdims=True))
    un = x * jnp.minimum(1.0, 1.0 / (n + 1e-7))
    items = disi[...] * (g0[...] + g1[...]) + b2[...]
    uv = jnp.sum(un * items, axis=1, keepdims=True)
    out[...] = jax.nn.sigmoid(uv)


_tc_f = pl.pallas_call(
    _tc_f_body,
    grid=(B // _ROWS_F,),
    in_specs=[
        pl.BlockSpec((_ROWS_F, D), lambda i: (i, 0)),
        pl.BlockSpec((_ROWS_F, D), lambda i: (i, 0)),
        pl.BlockSpec((_ROWS_F, D), lambda i: (i, 0)),
        pl.BlockSpec((_ROWS_F, D), lambda i: (i, 0)),
        pl.BlockSpec((1, D), lambda i: (0, 0)),
    ],
    out_specs=pl.BlockSpec((_ROWS_F, 1), lambda i: (i, 0)),
    out_shape=jax.ShapeDtypeStruct((B, 1), jnp.float32),
)


def kernel(u, i, edges, entitys_w, users_w, W1, b1, W2, b2):
    u = u.astype(jnp.int32)
    i = i.astype(jnp.int32)
    edges = edges.astype(jnp.int32)
    src2d = edges[0].reshape(NW, CH, K)
    dst2d = edges[1].reshape(NW, CH, K)
    u2d = u.reshape(NW, BPT)
    i2d = i.reshape(NW, BPT)
    z128 = jnp.zeros((N, D), jnp.float32)

    hist = _tc_deg(edges[1].reshape(E, 1))
    deg_col = hist.reshape(80 * D)[:N].reshape(N, 1)
    h1p, dis = _tc_b(entitys_w, W1, deg_col)
    accp = _sc_scatter_full(src2d, dst2d, h1p, z128)
    h2p = _tc_d(accp[0], accp[1], dis, b1.reshape(1, D), W2)
    accp2 = _sc_scatter_full(src2d, dst2d, h2p, z128)
    g, disi, urows = _sc_gather_items(accp2[0], accp2[1], dis, users_w, i2d, u2d)
    logit = _tc_f(urows, g[0], g[1], disi, b2.reshape(1, D))
    return logit.reshape(B)


# speedup vs baseline: 14.4436x; 1.1328x over previous
import functools

import jax
import jax.numpy as jnp
from jax import lax
from jax.experimental import pallas as pl
from jax.experimental.pallas import tpu as pltpu
from jax.experimental.pallas import tpu_sc as plsc

N = 10000
E = 320000
D = 128
B = 4096
NC = 2
NS = 16
NW = NC * NS
EPT = E // NW
K = 80
CH = EPT // K
CH1 = (CH + 1) // 2
BPT = B // NW

_mesh = plsc.VectorSubcoreMesh(core_axis_name="c", subcore_axis_name="s")


def _per_tile_rows(s, body):
    @pl.when(s < NS - 1)
    def _():
        body(s * 624, 624)

    @pl.when(s == NS - 1)
    def _():
        body(624 * (NS - 1), 640)


_EB = 4000


def _tc_deg_body(dst_ref, hist_ref):
    step = pl.program_id(0)
    d = dst_ref[...]
    r = d >> 7
    c = d & 127
    ohr = (r == lax.broadcasted_iota(jnp.int32, (1, 80), 1)).astype(jnp.bfloat16)
    ohc = (c == lax.broadcasted_iota(jnp.int32, (1, D), 1)).astype(jnp.bfloat16)
    h = lax.dot_general(ohr, ohc, (((0,), (0,)), ((), ())),
                        preferred_element_type=jnp.float32)

    @pl.when(step == 0)
    def _():
        hist_ref[...] = h

    @pl.when(step > 0)
    def _():
        hist_ref[...] += h


_tc_deg = pl.pallas_call(
    _tc_deg_body,
    grid=(E // _EB,),
    in_specs=[pl.BlockSpec((_EB, 1), lambda k: (k, 0))],
    out_specs=pl.BlockSpec((80, D), lambda k: (0, 0)),
    out_shape=jax.ShapeDtypeStruct((80, D), jnp.float32),
)


@functools.partial(
    pl.kernel,
    out_type=jax.ShapeDtypeStruct((NC, N, D), jnp.float32),
    mesh=_mesh,
    scratch_types=[
        pltpu.VMEM_SHARED((N, D), jnp.float32),
        pltpu.VMEM((CH, K), jnp.int32),
        pltpu.VMEM((2, 1, K), jnp.int32),
        pltpu.VMEM((2, 1, K), jnp.int32),
        pltpu.VMEM((K, D), jnp.float32),
        pltpu.VMEM((K, D), jnp.float32),
        pltpu.SemaphoreType.DMA,
        pltpu.SemaphoreType.DMA,
        pltpu.SemaphoreType.DMA,
        pltpu.SemaphoreType.DMA,
    ],
)
def _sc_scatter_full(src1d, dst2d, hp, z128,
                     accp_out,
                     acc_sh, dsti_v, ring_a, ring_b, rows_a, rows_b,
                     sem_a, sem_b, sem_ia, sem_ib):
    c = lax.axis_index("c")
    s = lax.axis_index("s")
    wid = c * NS + s
    base = wid * EPT

    @pl.when(c == 0)
    def _():
        _per_tile_rows(s, lambda o, n: pltpu.sync_copy(
            hp.at[pl.ds(o, n)], acc_sh.at[pl.ds(o, n)]))

    @pl.when(c == 1)
    def _():
        _per_tile_rows(s, lambda o, n: pltpu.sync_copy(
            z128.at[pl.ds(o, n)], acc_sh.at[pl.ds(o, n)]))

    pltpu.sync_copy(dst2d.at[wid], dsti_v)
    plsc.subcore_barrier()

    NP = (CH + 1) // 2
    pltpu.async_copy(src1d.at[pl.ds(base, K)], ring_a.at[0, 0], sem_ia)
    pltpu.async_copy(src1d.at[pl.ds(base + K, K)], ring_b.at[0, 0], sem_ib)

    @pl.loop(0, NP)
    def _(p):
        par = p & 1
        a = 2 * p
        b = a + 1
        pltpu.make_async_copy(src1d.at[pl.ds(base, K)], ring_a.at[par, 0],
                              sem_ia).wait()
        pltpu.async_copy(hp.at[ring_a.at[par, 0]], rows_a, sem_a)

        @pl.when(p + 1 < NP)
        def _():
            pltpu.async_copy(src1d.at[pl.ds(base + (a + 2) * K, K)],
                             ring_a.at[1 - par, 0], sem_ia)

        @pl.when(a + 3 < CH)
        def _():
            pltpu.async_copy(src1d.at[pl.ds(base + (a + 3) * K, K)],
                             ring_b.at[1 - par, 0], sem_ib)

        @pl.when(p > 0)
        def _():
            pltpu.make_async_copy(hp.at[ring_b.at[par, 0]], rows_b,
                                  sem_b).wait()
            pltpu.sync_copy(rows_b, acc_sh.at[dsti_v.at[b - 2]], add=True)

        pltpu.make_async_copy(hp.at[ring_a.at[par, 0]], rows_a,
                              sem_a).wait()
        pltpu.sync_copy(rows_a, acc_sh.at[dsti_v.at[a]], add=True)

        @pl.when(b < CH)
        def _():
            pltpu.make_async_copy(src1d.at[pl.ds(base + K, K)],
                                  ring_b.at[par, 0], sem_ib).wait()
            pltpu.async_copy(hp.at[ring_b.at[par, 0]], rows_b, sem_b)

    plsc.subcore_barrier()
    _per_tile_rows(s, lambda o, n: pltpu.sync_copy(
        acc_sh.at[pl.ds(o, n)], accp_out.at[c, pl.ds(o, n)]))


@functools.partial(
    pl.kernel,
    out_type=(
        jax.ShapeDtypeStruct((NC, B, D), jnp.float32),
        jax.ShapeDtypeStruct((B, D), jnp.float32),
        jax.ShapeDtypeStruct((B, D), jnp.float32),
    ),
    mesh=_mesh,
    scratch_types=[
        pltpu.VMEM((BPT,), jnp.int32),
        pltpu.VMEM((BPT // 4, D), jnp.float32),
        pltpu.SemaphoreType.DMA,
    ],
)
def _sc_gather_items(a0, a1, dis, users_w, i2d, u2d,
                     g_out, disi_out, urows_out,
                     iidx_v, irow_v, sem):
    c = lax.axis_index("c")
    s = lax.axis_index("s")
    wid = c * NS + s
    hb = BPT // 4
    pltpu.sync_copy(i2d.at[wid], iidx_v)
    for h in range(4):
        idx = iidx_v.at[pl.ds(h * hb, hb)]
        o = wid * BPT + h * hb
        pltpu.async_copy(a0.at[idx], irow_v, sem).wait()
        pltpu.sync_copy(irow_v, g_out.at[0, pl.ds(o, hb)])
        pltpu.async_copy(a1.at[idx], irow_v, sem).wait()
        pltpu.sync_copy(irow_v, g_out.at[1, pl.ds(o, hb)])
        pltpu.async_copy(dis.at[idx], irow_v, sem).wait()
        pltpu.sync_copy(irow_v, disi_out.at[pl.ds(o, hb)])
    pltpu.sync_copy(u2d.at[wid], iidx_v)
    for h in range(4):
        idx = iidx_v.at[pl.ds(h * hb, hb)]
        o = wid * BPT + h * hb
        pltpu.async_copy(users_w.at[idx], irow_v, sem).wait()
        pltpu.sync_copy(irow_v, urows_out.at[pl.ds(o, hb)])


_ROWS_B = 1000


def _tc_b_body(ent, w1, d0, h1p_out, dis_out):
    x = ent[...]
    n = jnp.sqrt(jnp.sum(x * x, axis=1, keepdims=True))
    x0 = x * jnp.minimum(1.0, 1.0 / (n + 1e-7))
    deg = d0[...] + 1.0
    dis = lax.rsqrt(deg)
    h1 = jnp.dot(x0, w1[...], preferred_element_type=jnp.float32)
    h1p_out[...] = h1 * dis
    dis_out[...] = jnp.broadcast_to(dis, (_ROWS_B, D))


_tc_b = pl.pallas_call(
    _tc_b_body,
    grid=(N // _ROWS_B,),
    in_specs=[
        pl.BlockSpec((_ROWS_B, D), lambda i: (i, 0)),
        pl.BlockSpec((D, D), lambda i: (0, 0)),
        pl.BlockSpec((_ROWS_B, 1), lambda i: (i, 0)),
    ],
    out_specs=(
        pl.BlockSpec((_ROWS_B, D), lambda i: (i, 0)),
        pl.BlockSpec((_ROWS_B, D), lambda i: (i, 0)),
    ),
    out_shape=(
        jax.ShapeDtypeStruct((N, D), jnp.float32),
        jax.ShapeDtypeStruct((N, D), jnp.float32),
    ),
)


def _tc_d_body(a0, a1, dis, b1, w2, h2p_out):
    x1 = jnp.maximum(dis[...] * (a0[...] + a1[...]) + b1[...], 0.0)
    h2 = jnp.dot(x1, w2[...], preferred_element_type=jnp.float32)
    h2p_out[...] = h2 * dis[...]


_tc_d = pl.pallas_call(
    _tc_d_body,
    grid=(N // _ROWS_B,),
    in_specs=[
        pl.BlockSpec((_ROWS_B, D), lambda i: (i, 0)),
        pl.BlockSpec((_ROWS_B, D), lambda i: (i, 0)),
        pl.BlockSpec((_ROWS_B, D), lambda i: (i, 0)),
        pl.BlockSpec((1, D), lambda i: (0, 0)),
        pl.BlockSpec((D, D), lambda i: (0, 0)),
    ],
    out_specs=pl.BlockSpec((_ROWS_B, D), lambda i: (i, 0)),
    out_shape=jax.ShapeDtypeStruct((N, D), jnp.float32),
)


_ROWS_F = 512


def _tc_f_body(ur, g0, g1, disi, b2, out):
    x = ur[...]
    n = jnp.sqrt(jnp.sum(x * x, axis=1, keepdims=True))
    un = x * jnp.minimum(1.0, 1.0 / (n + 1e-7))
    items = disi[...] * (g0[...] + g1[...]) + b2[...]
    uv = jnp.sum(un * items, axis=1, keepdims=True)
    out[...] = jax.nn.sigmoid(uv)


_tc_f = pl.pallas_call(
    _tc_f_body,
    grid=(B // _ROWS_F,),
    in_specs=[
        pl.BlockSpec((_ROWS_F, D), lambda i: (i, 0)),
        pl.BlockSpec((_ROWS_F, D), lambda i: (i, 0)),
        pl.BlockSpec((_ROWS_F, D), lambda i: (i, 0)),
        pl.BlockSpec((_ROWS_F, D), lambda i: (i, 0)),
        pl.BlockSpec((1, D), lambda i: (0, 0)),
    ],
    out_specs=pl.BlockSpec((_ROWS_F, 1), lambda i: (i, 0)),
    out_shape=jax.ShapeDtypeStruct((B, 1), jnp.float32),
)


def kernel(u, i, edges, entitys_w, users_w, W1, b1, W2, b2):
    u = u.astype(jnp.int32)
    i = i.astype(jnp.int32)
    edges = edges.astype(jnp.int32)
    src1d = edges[0]
    dst2d = edges[1].reshape(NW, CH, K)
    u2d = u.reshape(NW, BPT)
    i2d = i.reshape(NW, BPT)
    z128 = jnp.zeros((N, D), jnp.float32)

    hist = _tc_deg(edges[1].reshape(E, 1))
    deg_col = hist.reshape(80 * D)[:N].reshape(N, 1)
    h1p, dis = _tc_b(entitys_w, W1, deg_col)
    accp = _sc_scatter_full(src1d, dst2d, h1p, z128)
    h2p = _tc_d(accp[0], accp[1], dis, b1.reshape(1, D), W2)
    accp2 = _sc_scatter_full(src1d, dst2d, h2p, z128)
    g, disi, urows = _sc_gather_items(accp2[0], accp2[1], dis, users_w, i2d, u2d)
    logit = _tc_f(urows, g[0], g[1], disi, b2.reshape(1, D))
    return logit.reshape(B)


# speedup vs baseline: 14.6640x; 1.0153x over previous
import functools

import jax
import jax.numpy as jnp
from jax import lax
from jax.experimental import pallas as pl
from jax.experimental.pallas import tpu as pltpu
from jax.experimental.pallas import tpu_sc as plsc

N = 10000
E = 320000
D = 128
B = 4096
NC = 2
NS = 16
NW = NC * NS
EPT = E // NW
K = 80
CH = EPT // K
CH1 = (CH + 1) // 2
BPT = B // NW

_mesh = plsc.VectorSubcoreMesh(core_axis_name="c", subcore_axis_name="s")


def _per_tile_rows(s, body):
    @pl.when(s < NS - 1)
    def _():
        body(s * 624, 624)

    @pl.when(s == NS - 1)
    def _():
        body(624 * (NS - 1), 640)


_EB = 4000


def _tc_deg_body(dst_ref, hist_ref):
    step = pl.program_id(0)
    d = dst_ref[...]
    r = d >> 7
    c = d & 127
    ohr = (r == lax.broadcasted_iota(jnp.int32, (1, 80), 1)).astype(jnp.bfloat16)
    ohc = (c == lax.broadcasted_iota(jnp.int32, (1, D), 1)).astype(jnp.bfloat16)
    h = lax.dot_general(ohr, ohc, (((0,), (0,)), ((), ())),
                        preferred_element_type=jnp.float32)

    @pl.when(step == 0)
    def _():
        hist_ref[...] = h

    @pl.when(step > 0)
    def _():
        hist_ref[...] += h


_tc_deg = pl.pallas_call(
    _tc_deg_body,
    grid=(E // _EB,),
    in_specs=[pl.BlockSpec((_EB, 1), lambda k: (k, 0))],
    out_specs=pl.BlockSpec((80, D), lambda k: (0, 0)),
    out_shape=jax.ShapeDtypeStruct((80, D), jnp.float32),
)


@functools.partial(
    pl.kernel,
    out_type=jax.ShapeDtypeStruct((NC, N, D), jnp.float32),
    mesh=_mesh,
    scratch_types=[
        pltpu.VMEM_SHARED((N, D), jnp.float32),
        pltpu.VMEM((CH, K), jnp.int32),
        pltpu.VMEM((2, 1, K), jnp.int32),
        pltpu.VMEM((2, 1, K), jnp.int32),
        pltpu.VMEM((K, D), jnp.float32),
        pltpu.VMEM((K, D), jnp.float32),
        pltpu.SemaphoreType.DMA,
        pltpu.SemaphoreType.DMA,
        pltpu.SemaphoreType.DMA,
        pltpu.SemaphoreType.DMA,
        pltpu.SemaphoreType.DMA,
        pltpu.SemaphoreType.DMA,
    ],
)
def _sc_scatter_full(src1d, dst2d, hp, z128,
                     accp_out,
                     acc_sh, dsti_v, ring_a, ring_b, rows_a, rows_b,
                     sem_a, sem_b, sem_ia, sem_ib, sem_sa, sem_sb):
    c = lax.axis_index("c")
    s = lax.axis_index("s")
    wid = c * NS + s
    base = wid * EPT

    @pl.when(c == 0)
    def _():
        _per_tile_rows(s, lambda o, n: pltpu.sync_copy(
            hp.at[pl.ds(o, n)], acc_sh.at[pl.ds(o, n)]))

    @pl.when(c == 1)
    def _():
        _per_tile_rows(s, lambda o, n: pltpu.sync_copy(
            z128.at[pl.ds(o, n)], acc_sh.at[pl.ds(o, n)]))

    pltpu.sync_copy(dst2d.at[wid], dsti_v)
    plsc.subcore_barrier()

    NP = (CH + 1) // 2
    pltpu.async_copy(src1d.at[pl.ds(base, K)], ring_a.at[0, 0], sem_ia)
    pltpu.async_copy(src1d.at[pl.ds(base + K, K)], ring_b.at[0, 0], sem_ib)
    pltpu.make_async_copy(src1d.at[pl.ds(base, K)], ring_a.at[0, 0],
                          sem_ia).wait()
    pltpu.async_copy(hp.at[ring_a.at[0, 0]], rows_a, sem_a)

    @pl.loop(0, NP)
    def _(p):
        par = p & 1
        a = 2 * p
        b = a + 1
        pltpu.make_async_copy(hp.at[ring_a.at[par, 0]], rows_a, sem_a).wait()
        pltpu.async_copy(rows_a, acc_sh.at[dsti_v.at[a]], sem_sa, add=True)

        @pl.when(a + 2 < CH)
        def _():
            pltpu.async_copy(src1d.at[pl.ds(base + (a + 2) * K, K)],
                             ring_a.at[1 - par, 0], sem_ia)

        @pl.when(a + 3 < CH)
        def _():
            pltpu.async_copy(src1d.at[pl.ds(base + (a + 3) * K, K)],
                             ring_b.at[1 - par, 0], sem_ib)

        @pl.when(p > 0)
        def _():
            pltpu.make_async_copy(rows_b, acc_sh.at[dsti_v.at[b - 2]],
                                  sem_sb).wait()

        @pl.when(b < CH)
        def _():
            pltpu.make_async_copy(src1d.at[pl.ds(base + K, K)],
                                  ring_b.at[par, 0], sem_ib).wait()
            pltpu.async_copy(hp.at[ring_b.at[par, 0]], rows_b, sem_b)
            pltpu.make_async_copy(hp.at[ring_b.at[par, 0]], rows_b, sem_b).wait()
            pltpu.async_copy(rows_b, acc_sh.at[dsti_v.at[b]], sem_sb, add=True)

        @pl.when(a + 2 < CH)
        def _():
            pltpu.make_async_copy(rows_a, acc_sh.at[dsti_v.at[a]],
                                  sem_sa).wait()
            pltpu.make_async_copy(src1d.at[pl.ds(base, K)], ring_a.at[par, 0],
                                  sem_ia).wait()
            pltpu.async_copy(hp.at[ring_a.at[1 - par, 0]], rows_a, sem_a)

    pltpu.make_async_copy(rows_a, acc_sh.at[dsti_v.at[0]], sem_sa).wait()

    plsc.subcore_barrier()
    _per_tile_rows(s, lambda o, n: pltpu.sync_copy(
        acc_sh.at[pl.ds(o, n)], accp_out.at[c, pl.ds(o, n)]))


@functools.partial(
    pl.kernel,
    out_type=(
        jax.ShapeDtypeStruct((NC, B, D), jnp.float32),
        jax.ShapeDtypeStruct((B, D), jnp.float32),
        jax.ShapeDtypeStruct((B, D), jnp.float32),
    ),
    mesh=_mesh,
    scratch_types=[
        pltpu.VMEM((BPT,), jnp.int32),
        pltpu.VMEM((BPT // 4, D), jnp.float32),
        pltpu.SemaphoreType.DMA,
    ],
)
def _sc_gather_items(a0, a1, dis, users_w, i2d, u2d,
                     g_out, disi_out, urows_out,
                     iidx_v, irow_v, sem):
    c = lax.axis_index("c")
    s = lax.axis_index("s")
    wid = c * NS + s
    hb = BPT // 4
    pltpu.sync_copy(i2d.at[wid], iidx_v)
    for h in range(4):
        idx = iidx_v.at[pl.ds(h * hb, hb)]
        o = wid * BPT + h * hb
        pltpu.async_copy(a0.at[idx], irow_v, sem).wait()
        pltpu.sync_copy(irow_v, g_out.at[0, pl.ds(o, hb)])
        pltpu.async_copy(a1.at[idx], irow_v, sem).wait()
        pltpu.sync_copy(irow_v, g_out.at[1, pl.ds(o, hb)])
        pltpu.async_copy(dis.at[idx], irow_v, sem).wait()
        pltpu.sync_copy(irow_v, disi_out.at[pl.ds(o, hb)])
    pltpu.sync_copy(u2d.at[wid], iidx_v)
    for h in range(4):
        idx = iidx_v.at[pl.ds(h * hb, hb)]
        o = wid * BPT + h * hb
        pltpu.async_copy(users_w.at[idx], irow_v, sem).wait()
        pltpu.sync_copy(irow_v, urows_out.at[pl.ds(o, hb)])


_ROWS_B = 1000


def _tc_b_body(ent, w1, d0, h1p_out, dis_out):
    x = ent[...]
    n = jnp.sqrt(jnp.sum(x * x, axis=1, keepdims=True))
    x0 = x * jnp.minimum(1.0, 1.0 / (n + 1e-7))
    deg = d0[...] + 1.0
    dis = lax.rsqrt(deg)
    h1 = jnp.dot(x0, w1[...], preferred_element_type=jnp.float32)
    h1p_out[...] = h1 * dis
    dis_out[...] = jnp.broadcast_to(dis, (_ROWS_B, D))


_tc_b = pl.pallas_call(
    _tc_b_body,
    grid=(N // _ROWS_B,),
    in_specs=[
        pl.BlockSpec((_ROWS_B, D), lambda i: (i, 0)),
        pl.BlockSpec((D, D), lambda i: (0, 0)),
        pl.BlockSpec((_ROWS_B, 1), lambda i: (i, 0)),
    ],
    out_specs=(
        pl.BlockSpec((_ROWS_B, D), lambda i: (i, 0)),
        pl.BlockSpec((_ROWS_B, D), lambda i: (i, 0)),
    ),
    out_shape=(
        jax.ShapeDtypeStruct((N, D), jnp.float32),
        jax.ShapeDtypeStruct((N, D), jnp.float32),
    ),
)


def _tc_d_body(a0, a1, dis, b1, w2, h2p_out):
    x1 = jnp.maximum(dis[...] * (a0[...] + a1[...]) + b1[...], 0.0)
    h2 = jnp.dot(x1, w2[...], preferred_element_type=jnp.float32)
    h2p_out[...] = h2 * dis[...]


_tc_d = pl.pallas_call(
    _tc_d_body,
    grid=(N // _ROWS_B,),
    in_specs=[
        pl.BlockSpec((_ROWS_B, D), lambda i: (i, 0)),
        pl.BlockSpec((_ROWS_B, D), lambda i: (i, 0)),
        pl.BlockSpec((_ROWS_B, D), lambda i: (i, 0)),
        pl.BlockSpec((1, D), lambda i: (0, 0)),
        pl.BlockSpec((D, D), lambda i: (0, 0)),
    ],
    out_specs=pl.BlockSpec((_ROWS_B, D), lambda i: (i, 0)),
    out_shape=jax.ShapeDtypeStruct((N, D), jnp.float32),
)


_ROWS_F = 512


def _tc_f_body(ur, g0, g1, disi, b2, out):
    x = ur[...]
    n = jnp.sqrt(jnp.sum(x * x, axis=1, keepdims=True))
    un = x * jnp.minimum(1.0, 1.0 / (n + 1e-7))
    items = disi[...] * (g0[...] + g1[...]) + b2[...]
    uv = jnp.sum(un * items, axis=1, keepdims=True)
    out[...] = jax.nn.sigmoid(uv)


_tc_f = pl.pallas_call(
    _tc_f_body,
    grid=(B // _ROWS_F,),
    in_specs=[
        pl.BlockSpec((_ROWS_F, D), lambda i: (i, 0)),
        pl.BlockSpec((_ROWS_F, D), lambda i: (i, 0)),
        pl.BlockSpec((_ROWS_F, D), lambda i: (i, 0)),
        pl.BlockSpec((_ROWS_F, D), lambda i: (i, 0)),
        pl.BlockSpec((1, D), lambda i: (0, 0)),
    ],
    out_specs=pl.BlockSpec((_ROWS_F, 1), lambda i: (i, 0)),
    out_shape=jax.ShapeDtypeStruct((B, 1), jnp.float32),
)


def kernel(u, i, edges, entitys_w, users_w, W1, b1, W2, b2):
    u = u.astype(jnp.int32)
    i = i.astype(jnp.int32)
    edges = edges.astype(jnp.int32)
    src1d = edges[0]
    dst2d = edges[1].reshape(NW, CH, K)
    u2d = u.reshape(NW, BPT)
    i2d = i.reshape(NW, BPT)
    z128 = jnp.zeros((N, D), jnp.float32)

    hist = _tc_deg(edges[1].reshape(E, 1))
    deg_col = hist.reshape(80 * D)[:N].reshape(N, 1)
    h1p, dis = _tc_b(entitys_w, W1, deg_col)
    accp = _sc_scatter_full(src1d, dst2d, h1p, z128)
    h2p = _tc_d(accp[0], accp[1], dis, b1.reshape(1, D), W2)
    accp2 = _sc_scatter_full(src1d, dst2d, h2p, z128)
    g, disi, urows = _sc_gather_items(accp2[0], accp2[1], dis, users_w, i2d, u2d)
    logit = _tc_f(urows, g[0], g[1], disi, b2.reshape(1, D))
    return logit.reshape(B)


# speedup vs baseline: 15.2871x; 1.0425x over previous
import functools

import jax
import jax.numpy as jnp
from jax import lax
from jax.experimental import pallas as pl
from jax.experimental.pallas import tpu as pltpu
from jax.experimental.pallas import tpu_sc as plsc

N = 10000
E = 320000
D = 128
B = 4096
NC = 2
NS = 16
NW = NC * NS
EPT = E // NW
K = 80
CH = EPT // K
CH1 = (CH + 1) // 2
BPT = B // NW

_mesh = plsc.VectorSubcoreMesh(core_axis_name="c", subcore_axis_name="s")


def _per_tile_rows(s, body):
    @pl.when(s < NS - 1)
    def _():
        body(s * 624, 624)

    @pl.when(s == NS - 1)
    def _():
        body(624 * (NS - 1), 640)


_EB = 4000


def _tc_deg_body(dcol_ref, drow_ref, hist_ref):
    step = pl.program_id(0)
    dcol = dcol_ref[...]
    drow = drow_ref[...].reshape(1, _EB)
    ohr_t = ((drow >> 7) == lax.broadcasted_iota(jnp.int32, (80, 1), 0)
             ).astype(jnp.bfloat16)
    ohc = ((dcol & 127) == lax.broadcasted_iota(jnp.int32, (1, D), 1)
           ).astype(jnp.bfloat16)
    h = lax.dot_general(ohr_t, ohc, (((1,), (0,)), ((), ())),
                        preferred_element_type=jnp.float32)

    @pl.when(step == 0)
    def _():
        hist_ref[...] = h

    @pl.when(step > 0)
    def _():
        hist_ref[...] += h


_tc_deg = pl.pallas_call(
    _tc_deg_body,
    grid=(E // _EB,),
    in_specs=[
        pl.BlockSpec((_EB, 1), lambda k: (k, 0)),
        pl.BlockSpec((1, 1, _EB), lambda k: (k, 0, 0)),
    ],
    out_specs=pl.BlockSpec((80, D), lambda k: (0, 0)),
    out_shape=jax.ShapeDtypeStruct((80, D), jnp.float32),
)


@functools.partial(
    pl.kernel,
    out_type=jax.ShapeDtypeStruct((NC, N, D), jnp.float32),
    mesh=_mesh,
    scratch_types=[
        pltpu.VMEM_SHARED((N, D), jnp.float32),
        pltpu.VMEM((CH, K), jnp.int32),
        pltpu.VMEM((2, 1, K), jnp.int32),
        pltpu.VMEM((2, 1, K), jnp.int32),
        pltpu.VMEM((K, D), jnp.float32),
        pltpu.VMEM((K, D), jnp.float32),
        pltpu.SemaphoreType.DMA,
        pltpu.SemaphoreType.DMA,
        pltpu.SemaphoreType.DMA,
        pltpu.SemaphoreType.DMA,
        pltpu.SemaphoreType.DMA,
        pltpu.SemaphoreType.DMA,
    ],
)
def _sc_scatter_full(src1d, dst2d, hp, z128,
                     accp_out,
                     acc_sh, dsti_v, ring_a, ring_b, rows_a, rows_b,
                     sem_a, sem_b, sem_ia, sem_ib, sem_sa, sem_sb):
    c = lax.axis_index("c")
    s = lax.axis_index("s")
    wid = c * NS + s
    base = wid * EPT

    @pl.when(c == 0)
    def _():
        _per_tile_rows(s, lambda o, n: pltpu.sync_copy(
            hp.at[pl.ds(o, n)], acc_sh.at[pl.ds(o, n)]))

    @pl.when(c == 1)
    def _():
        _per_tile_rows(s, lambda o, n: pltpu.sync_copy(
            z128.at[pl.ds(o, n)], acc_sh.at[pl.ds(o, n)]))

    pltpu.sync_copy(dst2d.at[wid], dsti_v)
    plsc.subcore_barrier()

    NP = (CH + 1) // 2
    pltpu.async_copy(src1d.at[pl.ds(base, K)], ring_a.at[0, 0], sem_ia)
    pltpu.async_copy(src1d.at[pl.ds(base + K, K)], ring_b.at[0, 0], sem_ib)
    pltpu.make_async_copy(src1d.at[pl.ds(base, K)], ring_a.at[0, 0],
                          sem_ia).wait()
    pltpu.async_copy(hp.at[ring_a.at[0, 0]], rows_a, sem_a)

    @pl.loop(0, NP)
    def _(p):
        par = p & 1
        a = 2 * p
        b = a + 1
        pltpu.make_async_copy(hp.at[ring_a.at[par, 0]], rows_a, sem_a).wait()
        pltpu.async_copy(rows_a, acc_sh.at[dsti_v.at[a]], sem_sa, add=True)

        @pl.when(a + 2 < CH)
        def _():
            pltpu.async_copy(src1d.at[pl.ds(base + (a + 2) * K, K)],
                             ring_a.at[1 - par, 0], sem_ia)

        @pl.when(a + 3 < CH)
        def _():
            pltpu.async_copy(src1d.at[pl.ds(base + (a + 3) * K, K)],
                             ring_b.at[1 - par, 0], sem_ib)

        @pl.when(p > 0)
        def _():
            pltpu.make_async_copy(rows_b, acc_sh.at[dsti_v.at[b - 2]],
                                  sem_sb).wait()

        @pl.when(b < CH)
        def _():
            pltpu.make_async_copy(src1d.at[pl.ds(base + K, K)],
                                  ring_b.at[par, 0], sem_ib).wait()
            pltpu.async_copy(hp.at[ring_b.at[par, 0]], rows_b, sem_b)
            pltpu.make_async_copy(hp.at[ring_b.at[par, 0]], rows_b, sem_b).wait()
            pltpu.async_copy(rows_b, acc_sh.at[dsti_v.at[b]], sem_sb, add=True)

        @pl.when(a + 2 < CH)
        def _():
            pltpu.make_async_copy(rows_a, acc_sh.at[dsti_v.at[a]],
                                  sem_sa).wait()
            pltpu.make_async_copy(src1d.at[pl.ds(base, K)], ring_a.at[par, 0],
                                  sem_ia).wait()
            pltpu.async_copy(hp.at[ring_a.at[1 - par, 0]], rows_a, sem_a)

    pltpu.make_async_copy(rows_a, acc_sh.at[dsti_v.at[0]], sem_sa).wait()

    plsc.subcore_barrier()
    _per_tile_rows(s, lambda o, n: pltpu.sync_copy(
        acc_sh.at[pl.ds(o, n)], accp_out.at[c, pl.ds(o, n)]))


@functools.partial(
    pl.kernel,
    out_type=(
        jax.ShapeDtypeStruct((NC, B, D), jnp.float32),
        jax.ShapeDtypeStruct((B, D), jnp.float32),
        jax.ShapeDtypeStruct((B, D), jnp.float32),
    ),
    mesh=_mesh,
    scratch_types=[
        pltpu.VMEM((BPT,), jnp.int32),
        pltpu.VMEM((BPT // 4, D), jnp.float32),
        pltpu.SemaphoreType.DMA,
    ],
)
def _sc_gather_items(a0, a1, dis, users_w, i2d, u2d,
                     g_out, disi_out, urows_out,
                     iidx_v, irow_v, sem):
    c = lax.axis_index("c")
    s = lax.axis_index("s")
    wid = c * NS + s
    hb = BPT // 4
    pltpu.sync_copy(i2d.at[wid], iidx_v)
    for h in range(4):
        idx = iidx_v.at[pl.ds(h * hb, hb)]
        o = wid * BPT + h * hb
        pltpu.async_copy(a0.at[idx], irow_v, sem).wait()
        pltpu.sync_copy(irow_v, g_out.at[0, pl.ds(o, hb)])
        pltpu.async_copy(a1.at[idx], irow_v, sem).wait()
        pltpu.sync_copy(irow_v, g_out.at[1, pl.ds(o, hb)])
        pltpu.async_copy(dis.at[idx], irow_v, sem).wait()
        pltpu.sync_copy(irow_v, disi_out.at[pl.ds(o, hb)])
    pltpu.sync_copy(u2d.at[wid], iidx_v)
    for h in range(4):
        idx = iidx_v.at[pl.ds(h * hb, hb)]
        o = wid * BPT + h * hb
        pltpu.async_copy(users_w.at[idx], irow_v, sem).wait()
        pltpu.sync_copy(irow_v, urows_out.at[pl.ds(o, hb)])


_ROWS_B = 1000


def _tc_b_body(ent, w1, d0, h1p_out, dis_out):
    x = ent[...]
    n = jnp.sqrt(jnp.sum(x * x, axis=1, keepdims=True))
    x0 = x * jnp.minimum(1.0, 1.0 / (n + 1e-7))
    deg = d0[...] + 1.0
    dis = lax.rsqrt(deg)
    h1 = jnp.dot(x0, w1[...], preferred_element_type=jnp.float32)
    h1p_out[...] = h1 * dis
    dis_out[...] = jnp.broadcast_to(dis, (_ROWS_B, D))


_tc_b = pl.pallas_call(
    _tc_b_body,
    grid=(N // _ROWS_B,),
    in_specs=[
        pl.BlockSpec((_ROWS_B, D), lambda i: (i, 0)),
        pl.BlockSpec((D, D), lambda i: (0, 0)),
        pl.BlockSpec((_ROWS_B, 1), lambda i: (i, 0)),
    ],
    out_specs=(
        pl.BlockSpec((_ROWS_B, D), lambda i: (i, 0)),
        pl.BlockSpec((_ROWS_B, D), lambda i: (i, 0)),
    ),
    out_shape=(
        jax.ShapeDtypeStruct((N, D), jnp.float32),
        jax.ShapeDtypeStruct((N, D), jnp.float32),
    ),
)


def _tc_d_body(a0, a1, dis, b1, w2, h2p_out):
    x1 = jnp.maximum(dis[...] * (a0[...] + a1[...]) + b1[...], 0.0)
    h2 = jnp.dot(x1, w2[...], preferred_element_type=jnp.float32)
    h2p_out[...] = h2 * dis[...]


_tc_d = pl.pallas_call(
    _tc_d_body,
    grid=(N // _ROWS_B,),
    in_specs=[
        pl.BlockSpec((_ROWS_B, D), lambda i: (i, 0)),
        pl.BlockSpec((_ROWS_B, D), lambda i: (i, 0)),
        pl.BlockSpec((_ROWS_B, D), lambda i: (i, 0)),
        pl.BlockSpec((1, D), lambda i: (0, 0)),
        pl.BlockSpec((D, D), lambda i: (0, 0)),
    ],
    out_specs=pl.BlockSpec((_ROWS_B, D), lambda i: (i, 0)),
    out_shape=jax.ShapeDtypeStruct((N, D), jnp.float32),
)


_ROWS_F = 512


def _tc_f_body(ur, g0, g1, disi, b2, out):
    x = ur[...]
    n = jnp.sqrt(jnp.sum(x * x, axis=1, keepdims=True))
    un = x * jnp.minimum(1.0, 1.0 / (n + 1e-7))
    items = disi[...] * (g0[...] + g1[...]) + b2[...]
    uv = jnp.sum(un * items, axis=1, keepdims=True)
    out[...] = jax.nn.sigmoid(uv)


_tc_f = pl.pallas_call(
    _tc_f_body,
    grid=(B // _ROWS_F,),
    in_specs=[
        pl.BlockSpec((_ROWS_F, D), lambda i: (i, 0)),
        pl.BlockSpec((_ROWS_F, D), lambda i: (i, 0)),
        pl.BlockSpec((_ROWS_F, D), lambda i: (i, 0)),
        pl.BlockSpec((_ROWS_F, D), lambda i: (i, 0)),
        pl.BlockSpec((1, D), lambda i: (0, 0)),
    ],
    out_specs=pl.BlockSpec((_ROWS_F, 1), lambda i: (i, 0)),
    out_shape=jax.ShapeDtypeStruct((B, 1), jnp.float32),
)


def kernel(u, i, edges, entitys_w, users_w, W1, b1, W2, b2):
    u = u.astype(jnp.int32)
    i = i.astype(jnp.int32)
    edges = edges.astype(jnp.int32)
    src1d = edges[0]
    dst2d = edges[1].reshape(NW, CH, K)
    u2d = u.reshape(NW, BPT)
    i2d = i.reshape(NW, BPT)
    z128 = jnp.zeros((N, D), jnp.float32)

    hist = _tc_deg(edges[1].reshape(E, 1), edges[1].reshape(E // _EB, 1, _EB))
    deg_col = hist.reshape(80 * D)[:N].reshape(N, 1)
    h1p, dis = _tc_b(entitys_w, W1, deg_col)
    accp = _sc_scatter_full(src1d, dst2d, h1p, z128)
    h2p = _tc_d(accp[0], accp[1], dis, b1.reshape(1, D), W2)
    accp2 = _sc_scatter_full(src1d, dst2d, h2p, z128)
    g, disi, urows = _sc_gather_items(accp2[0], accp2[1], dis, users_w, i2d, u2d)
    logit = _tc_f(urows, g[0], g[1], disi, b2.reshape(1, D))
    return logit.reshape(B)


# speedup vs baseline: 19.9045x; 1.3020x over previous
import functools

import jax
import jax.numpy as jnp
from jax import lax
from jax.experimental import pallas as pl
from jax.experimental.pallas import tpu as pltpu
from jax.experimental.pallas import tpu_sc as plsc

N = 10000
E = 320000
D = 128
B = 4096
NC = 2
NS = 16
NW = NC * NS
EPT = E // NW
K = 80
CH = EPT // K
CH1 = (CH + 1) // 2
BPT = B // NW

_mesh = plsc.VectorSubcoreMesh(core_axis_name="c", subcore_axis_name="s")


def _per_tile_rows(s, body):
    @pl.when(s < NS - 1)
    def _():
        body(s * 624, 624)

    @pl.when(s == NS - 1)
    def _():
        body(624 * (NS - 1), 640)


_EB = 4000


def _tc_deg_body(d_ref, hist_ref):
    step = pl.program_id(0)
    d8 = d_ref[...]
    h = jnp.zeros((80, D), jnp.float32)
    for r8 in range(8):
        dr = lax.slice(d8, (r8, 0), (r8 + 1, _EB))
        u = ((dr >> 7) == lax.broadcasted_iota(jnp.int32, (80, 1), 0)
             ).astype(jnp.bfloat16)
        v = ((dr & 127) == lax.broadcasted_iota(jnp.int32, (D, 1), 0)
             ).astype(jnp.bfloat16)
        h = h + lax.dot_general(u, v, (((1,), (1,)), ((), ())),
                                preferred_element_type=jnp.float32)

    @pl.when(step == 0)
    def _():
        hist_ref[...] = h

    @pl.when(step > 0)
    def _():
        hist_ref[...] += h


_tc_deg = pl.pallas_call(
    _tc_deg_body,
    grid=(E // (8 * _EB),),
    in_specs=[pl.BlockSpec((8, _EB), lambda k: (k, 0))],
    out_specs=pl.BlockSpec((80, D), lambda k: (0, 0)),
    out_shape=jax.ShapeDtypeStruct((80, D), jnp.float32),
)


@functools.partial(
    pl.kernel,
    out_type=jax.ShapeDtypeStruct((NC, N, D), jnp.float32),
    mesh=_mesh,
    scratch_types=[
        pltpu.VMEM_SHARED((N, D), jnp.float32),
        pltpu.VMEM((CH, K), jnp.int32),
        pltpu.VMEM((2, 1, K), jnp.int32),
        pltpu.VMEM((2, 1, K), jnp.int32),
        pltpu.VMEM((K, D), jnp.float32),
        pltpu.VMEM((K, D), jnp.float32),
        pltpu.SemaphoreType.DMA,
        pltpu.SemaphoreType.DMA,
        pltpu.SemaphoreType.DMA,
        pltpu.SemaphoreType.DMA,
        pltpu.SemaphoreType.DMA,
        pltpu.SemaphoreType.DMA,
    ],
)
def _sc_scatter_full(src1d, dst2d, hp, z128,
                     accp_out,
                     acc_sh, dsti_v, ring_a, ring_b, rows_a, rows_b,
                     sem_a, sem_b, sem_ia, sem_ib, sem_sa, sem_sb):
    c = lax.axis_index("c")
    s = lax.axis_index("s")
    wid = c * NS + s
    base = wid * EPT

    @pl.when(c == 0)
    def _():
        _per_tile_rows(s, lambda o, n: pltpu.sync_copy(
            hp.at[pl.ds(o, n)], acc_sh.at[pl.ds(o, n)]))

    @pl.when(c == 1)
    def _():
        _per_tile_rows(s, lambda o, n: pltpu.sync_copy(
            z128.at[pl.ds(o, n)], acc_sh.at[pl.ds(o, n)]))

    pltpu.sync_copy(dst2d.at[wid], dsti_v)
    plsc.subcore_barrier()

    NP = (CH + 1) // 2
    pltpu.async_copy(src1d.at[pl.ds(base, K)], ring_a.at[0, 0], sem_ia)
    pltpu.async_copy(src1d.at[pl.ds(base + K, K)], ring_b.at[0, 0], sem_ib)
    pltpu.make_async_copy(src1d.at[pl.ds(base, K)], ring_a.at[0, 0],
                          sem_ia).wait()
    pltpu.async_copy(hp.at[ring_a.at[0, 0]], rows_a, sem_a)

    @pl.loop(0, NP)
    def _(p):
        par = p & 1
        a = 2 * p
        b = a + 1
        pltpu.make_async_copy(hp.at[ring_a.at[par, 0]], rows_a, sem_a).wait()
        pltpu.async_copy(rows_a, acc_sh.at[dsti_v.at[a]], sem_sa, add=True)

        @pl.when(a + 2 < CH)
        def _():
            pltpu.async_copy(src1d.at[pl.ds(base + (a + 2) * K, K)],
                             ring_a.at[1 - par, 0], sem_ia)

        @pl.when(a + 3 < CH)
        def _():
            pltpu.async_copy(src1d.at[pl.ds(base + (a + 3) * K, K)],
                             ring_b.at[1 - par, 0], sem_ib)

        @pl.when(p > 0)
        def _():
            pltpu.make_async_copy(rows_b, acc_sh.at[dsti_v.at[b - 2]],
                                  sem_sb).wait()

        @pl.when(b < CH)
        def _():
            pltpu.make_async_copy(src1d.at[pl.ds(base + K, K)],
                                  ring_b.at[par, 0], sem_ib).wait()
            pltpu.async_copy(hp.at[ring_b.at[par, 0]], rows_b, sem_b)
            pltpu.make_async_copy(hp.at[ring_b.at[par, 0]], rows_b, sem_b).wait()
            pltpu.async_copy(rows_b, acc_sh.at[dsti_v.at[b]], sem_sb, add=True)

        @pl.when(a + 2 < CH)
        def _():
            pltpu.make_async_copy(rows_a, acc_sh.at[dsti_v.at[a]],
                                  sem_sa).wait()
            pltpu.make_async_copy(src1d.at[pl.ds(base, K)], ring_a.at[par, 0],
                                  sem_ia).wait()
            pltpu.async_copy(hp.at[ring_a.at[1 - par, 0]], rows_a, sem_a)

    pltpu.make_async_copy(rows_a, acc_sh.at[dsti_v.at[0]], sem_sa).wait()

    plsc.subcore_barrier()
    _per_tile_rows(s, lambda o, n: pltpu.sync_copy(
        acc_sh.at[pl.ds(o, n)], accp_out.at[c, pl.ds(o, n)]))


@functools.partial(
    pl.kernel,
    out_type=(
        jax.ShapeDtypeStruct((NC, B, D), jnp.float32),
        jax.ShapeDtypeStruct((B, D), jnp.float32),
        jax.ShapeDtypeStruct((B, D), jnp.float32),
    ),
    mesh=_mesh,
    scratch_types=[
        pltpu.VMEM((BPT,), jnp.int32),
        pltpu.VMEM((BPT // 4, D), jnp.float32),
        pltpu.SemaphoreType.DMA,
    ],
)
def _sc_gather_items(a0, a1, dis, users_w, i2d, u2d,
                     g_out, disi_out, urows_out,
                     iidx_v, irow_v, sem):
    c = lax.axis_index("c")
    s = lax.axis_index("s")
    wid = c * NS + s
    hb = BPT // 4
    pltpu.sync_copy(i2d.at[wid], iidx_v)
    for h in range(4):
        idx = iidx_v.at[pl.ds(h * hb, hb)]
        o = wid * BPT + h * hb
        pltpu.async_copy(a0.at[idx], irow_v, sem).wait()
        pltpu.sync_copy(irow_v, g_out.at[0, pl.ds(o, hb)])
        pltpu.async_copy(a1.at[idx], irow_v, sem).wait()
        pltpu.sync_copy(irow_v, g_out.at[1, pl.ds(o, hb)])
        pltpu.async_copy(dis.at[idx], irow_v, sem).wait()
        pltpu.sync_copy(irow_v, disi_out.at[pl.ds(o, hb)])
    pltpu.sync_copy(u2d.at[wid], iidx_v)
    for h in range(4):
        idx = iidx_v.at[pl.ds(h * hb, hb)]
        o = wid * BPT + h * hb
        pltpu.async_copy(users_w.at[idx], irow_v, sem).wait()
        pltpu.sync_copy(irow_v, urows_out.at[pl.ds(o, hb)])


_ROWS_B = 1000


def _tc_b_body(ent, w1, d0, h1p_out, dis_out):
    x = ent[...]
    n = jnp.sqrt(jnp.sum(x * x, axis=1, keepdims=True))
    x0 = x * jnp.minimum(1.0, 1.0 / (n + 1e-7))
    deg = d0[...] + 1.0
    dis = lax.rsqrt(deg)
    h1 = jnp.dot(x0, w1[...], preferred_element_type=jnp.float32)
    h1p_out[...] = h1 * dis
    dis_out[...] = jnp.broadcast_to(dis, (_ROWS_B, D))


_tc_b = pl.pallas_call(
    _tc_b_body,
    grid=(N // _ROWS_B,),
    in_specs=[
        pl.BlockSpec((_ROWS_B, D), lambda i: (i, 0)),
        pl.BlockSpec((D, D), lambda i: (0, 0)),
        pl.BlockSpec((_ROWS_B, 1), lambda i: (i, 0)),
    ],
    out_specs=(
        pl.BlockSpec((_ROWS_B, D), lambda i: (i, 0)),
        pl.BlockSpec((_ROWS_B, D), lambda i: (i, 0)),
    ),
    out_shape=(
        jax.ShapeDtypeStruct((N, D), jnp.float32),
        jax.ShapeDtypeStruct((N, D), jnp.float32),
    ),
)


def _tc_d_body(a0, a1, dis, b1, w2, h2p_out):
    x1 = jnp.maximum(dis[...] * (a0[...] + a1[...]) + b1[...], 0.0)
    h2 = jnp.dot(x1, w2[...], preferred_element_type=jnp.float32)
    h2p_out[...] = h2 * dis[...]


_tc_d = pl.pallas_call(
    _tc_d_body,
    grid=(N // _ROWS_B,),
    in_specs=[
        pl.BlockSpec((_ROWS_B, D), lambda i: (i, 0)),
        pl.BlockSpec((_ROWS_B, D), lambda i: (i, 0)),
        pl.BlockSpec((_ROWS_B, D), lambda i: (i, 0)),
        pl.BlockSpec((1, D), lambda i: (0, 0)),
        pl.BlockSpec((D, D), lambda i: (0, 0)),
    ],
    out_specs=pl.BlockSpec((_ROWS_B, D), lambda i: (i, 0)),
    out_shape=jax.ShapeDtypeStruct((N, D), jnp.float32),
)


_ROWS_F = 512


def _tc_f_body(ur, g0, g1, disi, b2, out):
    x = ur[...]
    n = jnp.sqrt(jnp.sum(x * x, axis=1, keepdims=True))
    un = x * jnp.minimum(1.0, 1.0 / (n + 1e-7))
    items = disi[...] * (g0[...] + g1[...]) + b2[...]
    uv = jnp.sum(un * items, axis=1, keepdims=True)
    out[...] = jax.nn.sigmoid(uv)


_tc_f = pl.pallas_call(
    _tc_f_body,
    grid=(B // _ROWS_F,),
    in_specs=[
        pl.BlockSpec((_ROWS_F, D), lambda i: (i, 0)),
        pl.BlockSpec((_ROWS_F, D), lambda i: (i, 0)),
        pl.BlockSpec((_ROWS_F, D), lambda i: (i, 0)),
        pl.BlockSpec((_ROWS_F, D), lambda i: (i, 0)),
        pl.BlockSpec((1, D), lambda i: (0, 0)),
    ],
    out_specs=pl.BlockSpec((_ROWS_F, 1), lambda i: (i, 0)),
    out_shape=jax.ShapeDtypeStruct((B, 1), jnp.float32),
)


def kernel(u, i, edges, entitys_w, users_w, W1, b1, W2, b2):
    u = u.astype(jnp.int32)
    i = i.astype(jnp.int32)
    edges = edges.astype(jnp.int32)
    src1d = edges[0]
    dst2d = edges[1].reshape(NW, CH, K)
    u2d = u.reshape(NW, BPT)
    i2d = i.reshape(NW, BPT)
    z128 = jnp.zeros((N, D), jnp.float32)

    hist = _tc_deg(edges[1].reshape(E // _EB, _EB))
    deg_col = hist.reshape(80 * D)[:N].reshape(N, 1)
    h1p, dis = _tc_b(entitys_w, W1, deg_col)
    accp = _sc_scatter_full(src1d, dst2d, h1p, z128)
    h2p = _tc_d(accp[0], accp[1], dis, b1.reshape(1, D), W2)
    accp2 = _sc_scatter_full(src1d, dst2d, h2p, z128)
    g, disi, urows = _sc_gather_items(accp2[0], accp2[1], dis, users_w, i2d, u2d)
    logit = _tc_f(urows, g[0], g[1], disi, b2.reshape(1, D))
    return logit.reshape(B)


# speedup vs baseline: 20.0671x; 1.0082x over previous
import functools

import jax
import jax.numpy as jnp
from jax import lax
from jax.experimental import pallas as pl
from jax.experimental.pallas import tpu as pltpu
from jax.experimental.pallas import tpu_sc as plsc

N = 10000
E = 320000
D = 128
B = 4096
NC = 2
NS = 16
NW = NC * NS
EPT = E // NW
K = 80
CH = EPT // K
CH1 = (CH + 1) // 2
BPT = B // NW

_mesh = plsc.VectorSubcoreMesh(core_axis_name="c", subcore_axis_name="s")


def _per_tile_rows(s, body):
    @pl.when(s < NS - 1)
    def _():
        body(s * 624, 624)

    @pl.when(s == NS - 1)
    def _():
        body(624 * (NS - 1), 640)


_EB = 4000


def _tc_deg_body(d_ref, hist_ref):
    step = pl.program_id(0)
    d8 = d_ref[...]
    h = jnp.zeros((80, D), jnp.float32)
    for r8 in range(8):
        dr = lax.slice(d8, (r8, 0), (r8 + 1, _EB))
        u = ((dr >> 7) == lax.broadcasted_iota(jnp.int32, (80, 1), 0)
             ).astype(jnp.bfloat16)
        v = ((dr & 127) == lax.broadcasted_iota(jnp.int32, (D, 1), 0)
             ).astype(jnp.bfloat16)
        h = h + lax.dot_general(u, v, (((1,), (1,)), ((), ())),
                                preferred_element_type=jnp.float32)

    @pl.when(step == 0)
    def _():
        hist_ref[...] = h

    @pl.when(step > 0)
    def _():
        hist_ref[...] += h


_tc_deg = pl.pallas_call(
    _tc_deg_body,
    grid=(E // (8 * _EB),),
    in_specs=[pl.BlockSpec((8, _EB), lambda k: (k, 0))],
    out_specs=pl.BlockSpec((80, D), lambda k: (0, 0)),
    out_shape=jax.ShapeDtypeStruct((80, D), jnp.float32),
)


@functools.partial(
    pl.kernel,
    out_type=jax.ShapeDtypeStruct((NC, N, D), jnp.float32),
    mesh=_mesh,
    scratch_types=[
        pltpu.VMEM_SHARED((N, D), jnp.float32),
        pltpu.VMEM((2, 1, K), jnp.int32),
        pltpu.VMEM((2, 1, K), jnp.int32),
        pltpu.VMEM((2, 1, K), jnp.int32),
        pltpu.VMEM((2, 1, K), jnp.int32),
        pltpu.VMEM((K, D), jnp.float32),
        pltpu.VMEM((K, D), jnp.float32),
        pltpu.SemaphoreType.DMA,
        pltpu.SemaphoreType.DMA,
        pltpu.SemaphoreType.DMA,
        pltpu.SemaphoreType.DMA,
        pltpu.SemaphoreType.DMA,
        pltpu.SemaphoreType.DMA,
        pltpu.SemaphoreType.DMA,
        pltpu.SemaphoreType.DMA,
    ],
)
def _sc_scatter_full(src1d, dst1d, hp, z128,
                     accp_out,
                     acc_sh, ring_a, ring_b, ring_da, ring_db, rows_a, rows_b,
                     sem_a, sem_b, sem_ia, sem_ib, sem_da, sem_db,
                     sem_sa, sem_sb):
    c = lax.axis_index("c")
    s = lax.axis_index("s")
    wid = c * NS + s
    base = wid * EPT

    @pl.when(c == 0)
    def _():
        _per_tile_rows(s, lambda o, n: pltpu.sync_copy(
            hp.at[pl.ds(o, n)], acc_sh.at[pl.ds(o, n)]))

    @pl.when(c == 1)
    def _():
        _per_tile_rows(s, lambda o, n: pltpu.sync_copy(
            z128.at[pl.ds(o, n)], acc_sh.at[pl.ds(o, n)]))

    plsc.subcore_barrier()

    NP = (CH + 1) // 2
    pltpu.async_copy(src1d.at[pl.ds(base, K)], ring_a.at[0, 0], sem_ia)
    pltpu.async_copy(src1d.at[pl.ds(base + K, K)], ring_b.at[0, 0], sem_ib)
    pltpu.async_copy(dst1d.at[pl.ds(base, K)], ring_da.at[0, 0], sem_da)
    pltpu.async_copy(dst1d.at[pl.ds(base + K, K)], ring_db.at[0, 0], sem_db)
    pltpu.make_async_copy(src1d.at[pl.ds(base, K)], ring_a.at[0, 0],
                          sem_ia).wait()
    pltpu.async_copy(hp.at[ring_a.at[0, 0]], rows_a, sem_a)

    @pl.loop(0, NP)
    def _(p):
        par = p & 1
        a = 2 * p
        b = a + 1
        pltpu.make_async_copy(hp.at[ring_a.at[par, 0]], rows_a, sem_a).wait()
        pltpu.make_async_copy(dst1d.at[pl.ds(base, K)], ring_da.at[par, 0],
                              sem_da).wait()
        pltpu.async_copy(rows_a, acc_sh.at[ring_da.at[par, 0]], sem_sa,
                         add=True)

        @pl.when(a + 2 < CH)
        def _():
            pltpu.async_copy(src1d.at[pl.ds(base + (a + 2) * K, K)],
                             ring_a.at[1 - par, 0], sem_ia)
            pltpu.async_copy(dst1d.at[pl.ds(base + (a + 2) * K, K)],
                             ring_da.at[1 - par, 0], sem_da)

        @pl.when(a + 3 < CH)
        def _():
            pltpu.async_copy(src1d.at[pl.ds(base + (a + 3) * K, K)],
                             ring_b.at[1 - par, 0], sem_ib)

        @pl.when(p > 0)
        def _():
            pltpu.make_async_copy(rows_b, acc_sh.at[ring_db.at[par, 0]],
                                  sem_sb).wait()

        @pl.when(a + 3 < CH)
        def _():
            pltpu.async_copy(dst1d.at[pl.ds(base + (a + 3) * K, K)],
                             ring_db.at[1 - par, 0], sem_db)

        @pl.when(b < CH)
        def _():
            pltpu.make_async_copy(src1d.at[pl.ds(base + K, K)],
                                  ring_b.at[par, 0], sem_ib).wait()
            pltpu.async_copy(hp.at[ring_b.at[par, 0]], rows_b, sem_b)
            pltpu.make_async_copy(hp.at[ring_b.at[par, 0]], rows_b, sem_b).wait()
            pltpu.make_async_copy(dst1d.at[pl.ds(base + K, K)],
                                  ring_db.at[par, 0], sem_db).wait()
            pltpu.async_copy(rows_b, acc_sh.at[ring_db.at[par, 0]], sem_sb,
                             add=True)

        @pl.when(a + 2 < CH)
        def _():
            pltpu.make_async_copy(rows_a, acc_sh.at[ring_da.at[par, 0]],
                                  sem_sa).wait()
            pltpu.make_async_copy(src1d.at[pl.ds(base, K)], ring_a.at[par, 0],
                                  sem_ia).wait()
            pltpu.async_copy(hp.at[ring_a.at[1 - par, 0]], rows_a, sem_a)

    pltpu.make_async_copy(rows_a, acc_sh.at[ring_da.at[0, 0]], sem_sa).wait()

    plsc.subcore_barrier()
    _per_tile_rows(s, lambda o, n: pltpu.sync_copy(
        acc_sh.at[pl.ds(o, n)], accp_out.at[c, pl.ds(o, n)]))


@functools.partial(
    pl.kernel,
    out_type=(
        jax.ShapeDtypeStruct((NC, B, D), jnp.float32),
        jax.ShapeDtypeStruct((B, D), jnp.float32),
        jax.ShapeDtypeStruct((B, D), jnp.float32),
    ),
    mesh=_mesh,
    scratch_types=[
        pltpu.VMEM((BPT,), jnp.int32),
        pltpu.VMEM((BPT // 4, D), jnp.float32),
        pltpu.SemaphoreType.DMA,
    ],
)
def _sc_gather_items(a0, a1, dis, users_w, i2d, u2d,
                     g_out, disi_out, urows_out,
                     iidx_v, irow_v, sem):
    c = lax.axis_index("c")
    s = lax.axis_index("s")
    wid = c * NS + s
    hb = BPT // 4
    pltpu.sync_copy(i2d.at[wid], iidx_v)
    for h in range(4):
        idx = iidx_v.at[pl.ds(h * hb, hb)]
        o = wid * BPT + h * hb
        pltpu.async_copy(a0.at[idx], irow_v, sem).wait()
        pltpu.sync_copy(irow_v, g_out.at[0, pl.ds(o, hb)])
        pltpu.async_copy(a1.at[idx], irow_v, sem).wait()
        pltpu.sync_copy(irow_v, g_out.at[1, pl.ds(o, hb)])
        pltpu.async_copy(dis.at[idx], irow_v, sem).wait()
        pltpu.sync_copy(irow_v, disi_out.at[pl.ds(o, hb)])
    pltpu.sync_copy(u2d.at[wid], iidx_v)
    for h in range(4):
        idx = iidx_v.at[pl.ds(h * hb, hb)]
        o = wid * BPT + h * hb
        pltpu.async_copy(users_w.at[idx], irow_v, sem).wait()
        pltpu.sync_copy(irow_v, urows_out.at[pl.ds(o, hb)])


_ROWS_B = 1000


def _tc_b_body(ent, w1, d0, h1p_out, dis_out):
    x = ent[...]
    n = jnp.sqrt(jnp.sum(x * x, axis=1, keepdims=True))
    x0 = x * jnp.minimum(1.0, 1.0 / (n + 1e-7))
    deg = d0[...] + 1.0
    dis = lax.rsqrt(deg)
    h1 = jnp.dot(x0, w1[...], preferred_element_type=jnp.float32)
    h1p_out[...] = h1 * dis
    dis_out[...] = jnp.broadcast_to(dis, (_ROWS_B, D))


_tc_b = pl.pallas_call(
    _tc_b_body,
    grid=(N // _ROWS_B,),
    in_specs=[
        pl.BlockSpec((_ROWS_B, D), lambda i: (i, 0)),
        pl.BlockSpec((D, D), lambda i: (0, 0)),
        pl.BlockSpec((_ROWS_B, 1), lambda i: (i, 0)),
    ],
    out_specs=(
        pl.BlockSpec((_ROWS_B, D), lambda i: (i, 0)),
        pl.BlockSpec((_ROWS_B, D), lambda i: (i, 0)),
    ),
    out_shape=(
        jax.ShapeDtypeStruct((N, D), jnp.float32),
        jax.ShapeDtypeStruct((N, D), jnp.float32),
    ),
)


def _tc_d_body(a0, a1, dis, b1, w2, h2p_out):
    x1 = jnp.maximum(dis[...] * (a0[...] + a1[...]) + b1[...], 0.0)
    h2 = jnp.dot(x1, w2[...], preferred_element_type=jnp.float32)
    h2p_out[...] = h2 * dis[...]


_tc_d = pl.pallas_call(
    _tc_d_body,
    grid=(N // _ROWS_B,),
    in_specs=[
        pl.BlockSpec((_ROWS_B, D), lambda i: (i, 0)),
        pl.BlockSpec((_ROWS_B, D), lambda i: (i, 0)),
        pl.BlockSpec((_ROWS_B, D), lambda i: (i, 0)),
        pl.BlockSpec((1, D), lambda i: (0, 0)),
        pl.BlockSpec((D, D), lambda i: (0, 0)),
    ],
    out_specs=pl.BlockSpec((_ROWS_B, D), lambda i: (i, 0)),
    out_shape=jax.ShapeDtypeStruct((N, D), jnp.float32),
)


_ROWS_F = 512


def _tc_f_body(ur, g0, g1, disi, b2, out):
    x = ur[...]
    n = jnp.sqrt(jnp.sum(x * x, axis=1, keepdims=True))
    un = x * jnp.minimum(1.0, 1.0 / (n + 1e-7))
    items = disi[...] * (g0[...] + g1[...]) + b2[...]
    uv = jnp.sum(un * items, axis=1, keepdims=True)
    out[...] = jax.nn.sigmoid(uv)


_tc_f = pl.pallas_call(
    _tc_f_body,
    grid=(B // _ROWS_F,),
    in_specs=[
        pl.BlockSpec((_ROWS_F, D), lambda i: (i, 0)),
        pl.BlockSpec((_ROWS_F, D), lambda i: (i, 0)),
        pl.BlockSpec((_ROWS_F, D), lambda i: (i, 0)),
        pl.BlockSpec((_ROWS_F, D), lambda i: (i, 0)),
        pl.BlockSpec((1, D), lambda i: (0, 0)),
    ],
    out_specs=pl.BlockSpec((_ROWS_F, 1), lambda i: (i, 0)),
    out_shape=jax.ShapeDtypeStruct((B, 1), jnp.float32),
)


def kernel(u, i, edges, entitys_w, users_w, W1, b1, W2, b2):
    u = u.astype(jnp.int32)
    i = i.astype(jnp.int32)
    edges = edges.astype(jnp.int32)
    src1d = edges[0]
    dst1d = edges[1]
    u2d = u.reshape(NW, BPT)
    i2d = i.reshape(NW, BPT)
    z128 = jnp.zeros((N, D), jnp.float32)

    hist = _tc_deg(edges[1].reshape(E // _EB, _EB))
    deg_col = hist.reshape(80 * D)[:N].reshape(N, 1)
    h1p, dis = _tc_b(entitys_w, W1, deg_col)
    accp = _sc_scatter_full(src1d, dst1d, h1p, z128)
    h2p = _tc_d(accp[0], accp[1], dis, b1.reshape(1, D), W2)
    accp2 = _sc_scatter_full(src1d, dst1d, h2p, z128)
    g, disi, urows = _sc_gather_items(accp2[0], accp2[1], dis, users_w, i2d, u2d)
    logit = _tc_f(urows, g[0], g[1], disi, b2.reshape(1, D))
    return logit.reshape(B)
